# baseline (speedup 1.0000x reference)
import functools

import jax
import jax.numpy as jnp
from jax import lax
from jax.experimental import pallas as pl
from jax.experimental.pallas import tpu as pltpu
from jax.experimental.pallas import tpu_sc as plsc

N = 10000
D = 128
NC = 2
NS = 16
NW = NC * NS
NPAD = 10112
ROWS_PER_TILE = NPAD // NS
CHUNK = 128
NCHUNK0 = 60
NCHUNK1 = 98
EPW0 = NCHUNK0 * CHUNK
EPW1 = NCHUNK1 * CHUNK
EPAD = NS * (EPW0 + EPW1)

_MESH = plsc.VectorSubcoreMesh(core_axis_name="c", subcore_axis_name="s")


def _agg_body(with_cnt, *refs):
    if with_cnt:
        (h_hbm, src_hbm, dst_hbm, zrows_hbm, ones_hbm, psum_hbm, cnt_hbm,
         *bufs) = refs
    else:
        (h_hbm, src_hbm, dst_hbm, zrows_hbm, psum_hbm, *bufs) = refs
    (src0, dst0, rows0, acc_sh, gsem0) = bufs
    c = lax.axis_index("c")
    s = lax.axis_index("s")
    rs = s * ROWS_PER_TILE
    base0 = jnp.where(c == 0, s * EPW0, NS * EPW0 + s * EPW1)
    n_my = jnp.where(c == 0, NCHUNK0, NCHUNK1)

    pltpu.sync_copy(zrows_hbm, acc_sh.at[pl.ds(rs, ROWS_PER_TILE)])
    plsc.subcore_barrier()

    def step(i, carry):
        base = base0 + i * CHUNK
        pltpu.sync_copy(src_hbm.at[pl.ds(base, CHUNK)], src0)
        pltpu.sync_copy(dst_hbm.at[pl.ds(base, CHUNK)], dst0)
        pltpu.async_copy(h_hbm.at[src0], rows0, gsem0).wait()
        pltpu.sync_copy(rows0, acc_sh.at[dst0], add=True)
        return carry

    lax.fori_loop(0, n_my, step, 0)
    plsc.subcore_barrier()

    pltpu.sync_copy(acc_sh.at[pl.ds(rs, ROWS_PER_TILE)],
                    psum_hbm.at[c, pl.ds(rs, ROWS_PER_TILE)])

    if with_cnt:
        plsc.subcore_barrier()
        pltpu.sync_copy(zrows_hbm, acc_sh.at[pl.ds(rs, ROWS_PER_TILE)])
        pltpu.sync_copy(ones_hbm, rows0)
        plsc.subcore_barrier()

        def cstep(i, carry):
            base = base0 + i * CHUNK
            pltpu.sync_copy(dst_hbm.at[pl.ds(base, CHUNK)], dst0)
            pltpu.sync_copy(rows0, acc_sh.at[dst0], add=True)
            return carry

        lax.fori_loop(0, n_my, cstep, 0)
        plsc.subcore_barrier()
        pltpu.sync_copy(acc_sh.at[pl.ds(rs, ROWS_PER_TILE)],
                        cnt_hbm.at[c, pl.ds(rs, ROWS_PER_TILE)])


def _make_agg(with_cnt):
    out_type = [jax.ShapeDtypeStruct((NC, NPAD, D), jnp.float32)]
    if with_cnt:
        out_type.append(jax.ShapeDtypeStruct((NC, NPAD, D), jnp.float32))
    scratch = [
        pltpu.VMEM((CHUNK,), jnp.int32),
        pltpu.VMEM((CHUNK,), jnp.int32),
        pltpu.VMEM((CHUNK, D), jnp.float32),
        pltpu.VMEM_SHARED((NPAD, D), jnp.float32),
        pltpu.SemaphoreType.DMA,
    ]
    return pl.kernel(
        functools.partial(_agg_body, with_cnt),
        out_type=tuple(out_type) if with_cnt else out_type[0],
        mesh=_MESH,
        scratch_types=scratch,
        name="sage_edge_agg" + ("_cnt" if with_cnt else ""),
    )


_ROWS_BLK = 1024


def _recip_body(cnt_ref, out_ref):
    cnt = cnt_ref[0, :, 0:1] + cnt_ref[1, :, 0:1]
    out_ref[...] = 1.0 / jnp.maximum(cnt, 1.0)


_recip_call = pl.pallas_call(
    _recip_body,
    grid=(pl.cdiv(N, _ROWS_BLK),),
    in_specs=[pl.BlockSpec((NC, _ROWS_BLK, D), lambda i: (0, i, 0))],
    out_specs=pl.BlockSpec((_ROWS_BLK, 1), lambda i: (i, 0)),
    out_shape=jax.ShapeDtypeStruct((N, 1), jnp.float32),
    name="sage_recip_cnt",
)


def _layer_body(relu_res, p_ref, recip_ref, h_ref, wl_ref, bl_ref, wr_ref,
                out_ref):
    mean = (p_ref[0] + p_ref[1]) * recip_ref[...]
    acc = jnp.dot(mean, wl_ref[...], preferred_element_type=jnp.float32)
    acc = acc + jnp.dot(h_ref[...], wr_ref[...], preferred_element_type=jnp.float32)
    acc = acc + bl_ref[...]
    if relu_res:
        acc = jnp.maximum(acc, 0.0) + h_ref[...]
    out_ref[...] = acc


def _make_layer(relu_res):
    return pl.pallas_call(
        functools.partial(_layer_body, relu_res),
        grid=(pl.cdiv(N, _ROWS_BLK),),
        in_specs=[
            pl.BlockSpec((NC, _ROWS_BLK, D), lambda i: (0, i, 0)),
            pl.BlockSpec((_ROWS_BLK, 1), lambda i: (i, 0)),
            pl.BlockSpec((_ROWS_BLK, D), lambda i: (i, 0)),
            pl.BlockSpec((D, D), lambda i: (0, 0)),
            pl.BlockSpec((1, D), lambda i: (0, 0)),
            pl.BlockSpec((D, D), lambda i: (0, 0)),
        ],
        out_specs=pl.BlockSpec((_ROWS_BLK, D), lambda i: (i, 0)),
        out_shape=jax.ShapeDtypeStruct((N, D), jnp.float32),
        name="sage_dense" + ("_relu_res" if relu_res else ""),
    )


def kernel(x, edge_index, W0l, b0l, W0r, W1l, b1l, W1r, W2l, b2l, W2r):
    e = edge_index.shape[1]
    pad = EPAD - e
    src = jnp.concatenate([edge_index[0], jnp.zeros((pad,), jnp.int32)])
    dst = jnp.concatenate([edge_index[1], jnp.full((pad,), N, jnp.int32)])
    zrows = jnp.zeros((ROWS_PER_TILE, D), jnp.float32)
    ones_rows = jnp.ones((CHUNK, D), jnp.float32)

    agg_cnt = _make_agg(True)
    agg = _make_agg(False)
    layer_mid = _make_layer(True)
    layer_last = _make_layer(False)

    p0, cnt = agg_cnt(x, src, dst, zrows, ones_rows)
    recip = _recip_call(cnt)
    h = layer_mid(p0, recip, x, W0l, b0l.reshape(1, D), W0r)
    p1 = agg(h, src, dst, zrows)
    h = layer_mid(p1, recip, h, W1l, b1l.reshape(1, D), W1r)
    p2 = agg(h, src, dst, zrows)
    h = layer_last(p2, recip, h, W2l, b2l.reshape(1, D), W2r)
    return h


# speedup vs baseline: 1.2784x; 1.2784x over previous
import functools

import jax
import jax.numpy as jnp
from jax import lax
from jax.experimental import pallas as pl
from jax.experimental.pallas import tpu as pltpu
from jax.experimental.pallas import tpu_sc as plsc

N = 10000
D = 128
NC = 2
NS = 16
NW = NC * NS
NPAD = 10112
ROWS_PER_TILE = NPAD // NS
CHUNK = 128
NCHUNK0 = 96
NCHUNK1 = 62
EPW0 = NCHUNK0 * CHUNK
EPW1 = NCHUNK1 * CHUNK
EPAD = NS * (EPW0 + EPW1)

_MESH = plsc.VectorSubcoreMesh(core_axis_name="c", subcore_axis_name="s")


def _agg_body(with_cnt, *refs):
    if with_cnt:
        (h_hbm, src_hbm, dst_hbm, zrows_hbm, ones_hbm, psum_hbm, cnt_hbm,
         *bufs) = refs
    else:
        (h_hbm, src_hbm, dst_hbm, zrows_hbm, psum_hbm, *bufs) = refs
    (src0, dst0, rows0, acc_sh, gsem0) = bufs
    c = lax.axis_index("c")
    s = lax.axis_index("s")
    rs = s * ROWS_PER_TILE
    base0 = jnp.where(c == 0, s * EPW0, NS * EPW0 + s * EPW1)
    n_my = jnp.where(c == 0, NCHUNK0, NCHUNK1)

    pltpu.sync_copy(zrows_hbm, acc_sh.at[pl.ds(rs, ROWS_PER_TILE)])
    plsc.subcore_barrier()

    def step(i, carry):
        base = base0 + i * CHUNK
        pltpu.sync_copy(src_hbm.at[pl.ds(base, CHUNK)], src0)
        pltpu.sync_copy(dst_hbm.at[pl.ds(base, CHUNK)], dst0)
        pltpu.async_copy(h_hbm.at[src0], rows0, gsem0).wait()
        pltpu.sync_copy(rows0, acc_sh.at[dst0], add=True)
        return carry

    lax.fori_loop(0, n_my, step, 0)
    plsc.subcore_barrier()

    pltpu.sync_copy(acc_sh.at[pl.ds(rs, ROWS_PER_TILE)],
                    psum_hbm.at[c, pl.ds(rs, ROWS_PER_TILE)])

    if with_cnt:
        plsc.subcore_barrier()
        pltpu.sync_copy(zrows_hbm, acc_sh.at[pl.ds(rs, ROWS_PER_TILE)])
        pltpu.sync_copy(ones_hbm, rows0)
        plsc.subcore_barrier()

        def cstep(i, carry):
            base = base0 + i * CHUNK
            pltpu.sync_copy(dst_hbm.at[pl.ds(base, CHUNK)], dst0)
            pltpu.sync_copy(rows0, acc_sh.at[dst0], add=True)
            return carry

        lax.fori_loop(0, n_my, cstep, 0)
        plsc.subcore_barrier()
        pltpu.sync_copy(acc_sh.at[pl.ds(rs, ROWS_PER_TILE)],
                        cnt_hbm.at[c, pl.ds(rs, ROWS_PER_TILE)])


def _make_agg(with_cnt):
    out_type = [jax.ShapeDtypeStruct((NC, NPAD, D), jnp.float32)]
    if with_cnt:
        out_type.append(jax.ShapeDtypeStruct((NC, NPAD, D), jnp.float32))
    scratch = [
        pltpu.VMEM((CHUNK,), jnp.int32),
        pltpu.VMEM((CHUNK,), jnp.int32),
        pltpu.VMEM((CHUNK, D), jnp.float32),
        pltpu.VMEM_SHARED((NPAD, D), jnp.float32),
        pltpu.SemaphoreType.DMA,
    ]
    return pl.kernel(
        functools.partial(_agg_body, with_cnt),
        out_type=tuple(out_type) if with_cnt else out_type[0],
        mesh=_MESH,
        scratch_types=scratch,
        name="sage_edge_agg" + ("_cnt" if with_cnt else ""),
    )


_ROWS_BLK = 1024


def _recip_body(cnt_ref, out_ref):
    cnt = cnt_ref[0, :, 0:1] + cnt_ref[1, :, 0:1]
    out_ref[...] = 1.0 / jnp.maximum(cnt, 1.0)


_recip_call = pl.pallas_call(
    _recip_body,
    grid=(pl.cdiv(N, _ROWS_BLK),),
    in_specs=[pl.BlockSpec((NC, _ROWS_BLK, D), lambda i: (0, i, 0))],
    out_specs=pl.BlockSpec((_ROWS_BLK, 1), lambda i: (i, 0)),
    out_shape=jax.ShapeDtypeStruct((N, 1), jnp.float32),
    name="sage_recip_cnt",
)


def _layer_body(relu_res, p_ref, recip_ref, h_ref, wl_ref, bl_ref, wr_ref,
                out_ref):
    mean = (p_ref[0] + p_ref[1]) * recip_ref[...]
    acc = jnp.dot(mean, wl_ref[...], preferred_element_type=jnp.float32)
    acc = acc + jnp.dot(h_ref[...], wr_ref[...], preferred_element_type=jnp.float32)
    acc = acc + bl_ref[...]
    if relu_res:
        acc = jnp.maximum(acc, 0.0) + h_ref[...]
    out_ref[...] = acc


def _make_layer(relu_res):
    return pl.pallas_call(
        functools.partial(_layer_body, relu_res),
        grid=(pl.cdiv(N, _ROWS_BLK),),
        in_specs=[
            pl.BlockSpec((NC, _ROWS_BLK, D), lambda i: (0, i, 0)),
            pl.BlockSpec((_ROWS_BLK, 1), lambda i: (i, 0)),
            pl.BlockSpec((_ROWS_BLK, D), lambda i: (i, 0)),
            pl.BlockSpec((D, D), lambda i: (0, 0)),
            pl.BlockSpec((1, D), lambda i: (0, 0)),
            pl.BlockSpec((D, D), lambda i: (0, 0)),
        ],
        out_specs=pl.BlockSpec((_ROWS_BLK, D), lambda i: (i, 0)),
        out_shape=jax.ShapeDtypeStruct((N, D), jnp.float32),
        name="sage_dense" + ("_relu_res" if relu_res else ""),
    )


def kernel(x, edge_index, W0l, b0l, W0r, W1l, b1l, W1r, W2l, b2l, W2r):
    e = edge_index.shape[1]
    pad = EPAD - e
    src = jnp.concatenate([edge_index[0], jnp.zeros((pad,), jnp.int32)])
    dst = jnp.concatenate([edge_index[1], jnp.full((pad,), N, jnp.int32)])
    zrows = jnp.zeros((ROWS_PER_TILE, D), jnp.float32)
    ones_rows = jnp.ones((CHUNK, D), jnp.float32)

    agg_cnt = _make_agg(True)
    agg = _make_agg(False)
    layer_mid = _make_layer(True)
    layer_last = _make_layer(False)

    p0, cnt = agg_cnt(x, src, dst, zrows, ones_rows)
    recip = _recip_call(cnt)
    h = layer_mid(p0, recip, x, W0l, b0l.reshape(1, D), W0r)
    p1 = agg(h, src, dst, zrows)
    h = layer_mid(p1, recip, h, W1l, b1l.reshape(1, D), W1r)
    p2 = agg(h, src, dst, zrows)
    h = layer_last(p2, recip, h, W2l, b2l.reshape(1, D), W2r)
    return h


# speedup vs baseline: 1.2872x; 1.0069x over previous
import functools

import jax
import jax.numpy as jnp
from jax import lax
from jax.experimental import pallas as pl
from jax.experimental.pallas import tpu as pltpu
from jax.experimental.pallas import tpu_sc as plsc

N = 10000
D = 128
NC = 2
NS = 16
NW = NC * NS
NPAD = 10112
ROWS_PER_TILE = NPAD // NS
CHUNK = 128
NCHUNK0 = 98
NCHUNK1 = 60
EPW0 = NCHUNK0 * CHUNK
EPW1 = NCHUNK1 * CHUNK
EPAD = NS * (EPW0 + EPW1)

_MESH = plsc.VectorSubcoreMesh(core_axis_name="c", subcore_axis_name="s")


def _agg_body(with_cnt, *refs):
    if with_cnt:
        (h_hbm, src_hbm, dst_hbm, zrows_hbm, ones_hbm, psum_hbm, cnt_hbm,
         *bufs) = refs
    else:
        (h_hbm, src_hbm, dst_hbm, zrows_hbm, psum_hbm, *bufs) = refs
    (src0, dst0, rows0, acc_sh, gsem0) = bufs
    c = lax.axis_index("c")
    s = lax.axis_index("s")
    rs = s * ROWS_PER_TILE
    base0 = jnp.where(c == 0, s * EPW0, NS * EPW0 + s * EPW1)
    n_my = jnp.where(c == 0, NCHUNK0, NCHUNK1)

    pltpu.sync_copy(zrows_hbm, acc_sh.at[pl.ds(rs, ROWS_PER_TILE)])
    plsc.subcore_barrier()

    def step(i, carry):
        base = base0 + i * CHUNK
        pltpu.sync_copy(src_hbm.at[pl.ds(base, CHUNK)], src0)
        pltpu.sync_copy(dst_hbm.at[pl.ds(base, CHUNK)], dst0)
        pltpu.async_copy(h_hbm.at[src0], rows0, gsem0).wait()
        pltpu.sync_copy(rows0, acc_sh.at[dst0], add=True)
        return carry

    lax.fori_loop(0, n_my, step, 0)
    plsc.subcore_barrier()

    pltpu.sync_copy(acc_sh.at[pl.ds(rs, ROWS_PER_TILE)],
                    psum_hbm.at[c, pl.ds(rs, ROWS_PER_TILE)])

    if with_cnt:
        plsc.subcore_barrier()
        pltpu.sync_copy(zrows_hbm, acc_sh.at[pl.ds(rs, ROWS_PER_TILE)])
        pltpu.sync_copy(ones_hbm, rows0)
        plsc.subcore_barrier()

        def cstep(i, carry):
            base = base0 + i * CHUNK
            pltpu.sync_copy(dst_hbm.at[pl.ds(base, CHUNK)], dst0)
            pltpu.sync_copy(rows0, acc_sh.at[dst0], add=True)
            return carry

        lax.fori_loop(0, n_my, cstep, 0)
        plsc.subcore_barrier()
        pltpu.sync_copy(acc_sh.at[pl.ds(rs, ROWS_PER_TILE)],
                        cnt_hbm.at[c, pl.ds(rs, ROWS_PER_TILE)])


def _make_agg(with_cnt):
    out_type = [jax.ShapeDtypeStruct((NC, NPAD, D), jnp.float32)]
    if with_cnt:
        out_type.append(jax.ShapeDtypeStruct((NC, NPAD, D), jnp.float32))
    scratch = [
        pltpu.VMEM((CHUNK,), jnp.int32),
        pltpu.VMEM((CHUNK,), jnp.int32),
        pltpu.VMEM((CHUNK, D), jnp.float32),
        pltpu.VMEM_SHARED((NPAD, D), jnp.float32),
        pltpu.SemaphoreType.DMA,
    ]
    return pl.kernel(
        functools.partial(_agg_body, with_cnt),
        out_type=tuple(out_type) if with_cnt else out_type[0],
        mesh=_MESH,
        scratch_types=scratch,
        name="sage_edge_agg" + ("_cnt" if with_cnt else ""),
    )


_ROWS_BLK = 1024


def _recip_body(cnt_ref, out_ref):
    cnt = cnt_ref[0, :, 0:1] + cnt_ref[1, :, 0:1]
    out_ref[...] = 1.0 / jnp.maximum(cnt, 1.0)


_recip_call = pl.pallas_call(
    _recip_body,
    grid=(pl.cdiv(N, _ROWS_BLK),),
    in_specs=[pl.BlockSpec((NC, _ROWS_BLK, D), lambda i: (0, i, 0))],
    out_specs=pl.BlockSpec((_ROWS_BLK, 1), lambda i: (i, 0)),
    out_shape=jax.ShapeDtypeStruct((N, 1), jnp.float32),
    name="sage_recip_cnt",
)


def _layer_body(relu_res, p_ref, recip_ref, h_ref, wl_ref, bl_ref, wr_ref,
                out_ref):
    mean = (p_ref[0] + p_ref[1]) * recip_ref[...]
    acc = jnp.dot(mean, wl_ref[...], preferred_element_type=jnp.float32)
    acc = acc + jnp.dot(h_ref[...], wr_ref[...], preferred_element_type=jnp.float32)
    acc = acc + bl_ref[...]
    if relu_res:
        acc = jnp.maximum(acc, 0.0) + h_ref[...]
    out_ref[...] = acc


def _make_layer(relu_res):
    return pl.pallas_call(
        functools.partial(_layer_body, relu_res),
        grid=(pl.cdiv(N, _ROWS_BLK),),
        in_specs=[
            pl.BlockSpec((NC, _ROWS_BLK, D), lambda i: (0, i, 0)),
            pl.BlockSpec((_ROWS_BLK, 1), lambda i: (i, 0)),
            pl.BlockSpec((_ROWS_BLK, D), lambda i: (i, 0)),
            pl.BlockSpec((D, D), lambda i: (0, 0)),
            pl.BlockSpec((1, D), lambda i: (0, 0)),
            pl.BlockSpec((D, D), lambda i: (0, 0)),
        ],
        out_specs=pl.BlockSpec((_ROWS_BLK, D), lambda i: (i, 0)),
        out_shape=jax.ShapeDtypeStruct((N, D), jnp.float32),
        name="sage_dense" + ("_relu_res" if relu_res else ""),
    )


def kernel(x, edge_index, W0l, b0l, W0r, W1l, b1l, W1r, W2l, b2l, W2r):
    e = edge_index.shape[1]
    pad = EPAD - e
    src = jnp.concatenate([edge_index[0], jnp.zeros((pad,), jnp.int32)])
    dst = jnp.concatenate([edge_index[1], jnp.full((pad,), N, jnp.int32)])
    zrows = jnp.zeros((ROWS_PER_TILE, D), jnp.float32)
    ones_rows = jnp.ones((CHUNK, D), jnp.float32)

    agg_cnt = _make_agg(True)
    agg = _make_agg(False)
    layer_mid = _make_layer(True)
    layer_last = _make_layer(False)

    p0, cnt = agg_cnt(x, src, dst, zrows, ones_rows)
    recip = _recip_call(cnt)
    h = layer_mid(p0, recip, x, W0l, b0l.reshape(1, D), W0r)
    p1 = agg(h, src, dst, zrows)
    h = layer_mid(p1, recip, h, W1l, b1l.reshape(1, D), W1r)
    p2 = agg(h, src, dst, zrows)
    h = layer_last(p2, recip, h, W2l, b2l.reshape(1, D), W2r)
    return h
